# baseline (speedup 1.0000x reference)
import functools

import jax
import jax.numpy as jnp
from jax import lax
from jax.experimental import pallas as pl
from jax.experimental.pallas import tpu as pltpu
from jax.experimental.pallas import tpu_sc as plsc

N = 10000
D = 128
E = 320000
G = 64
C = 10

NC, NS = 2, 16
DH = 64
JG = 160
EPT = JG * 128
E_PAD = NS * EPT
NPAD = 10240
ZPT = NPAD // NS
CH = 32
NB = 8


def _agg_body(hp_hbm, src_hbm, dst_hbm, out0, out1,
              src_cv, dst_cv, rows_v, tab_sh, acc_sh, gsems, ssems, isems):
    cid = lax.axis_index("c")
    sid = lax.axis_index("s")

    zbase = sid * ZPT
    for g in range(5):
        sl = pl.ds(zbase + g * 128, 128)
        pltpu.async_copy(hp_hbm.at[cid, sl], rows_v.at[g], gsems.at[g])
    for g in range(5):
        sl = pl.ds(zbase + g * 128, 128)
        pltpu.make_async_copy(hp_hbm.at[cid, sl], rows_v.at[g], gsems.at[g]).wait()
        pltpu.async_copy(rows_v.at[g], tab_sh.at[sl], ssems.at[g])
        pltpu.async_copy(rows_v.at[g], acc_sh.at[sl], ssems.at[g + 1])
    for g in range(5):
        sl = pl.ds(zbase + g * 128, 128)
        pltpu.make_async_copy(rows_v.at[g], tab_sh.at[sl], ssems.at[g]).wait()
        pltpu.make_async_copy(rows_v.at[g], acc_sh.at[sl], ssems.at[g + 1]).wait()
    plsc.subcore_barrier()

    def idx_load(i, p):
        pltpu.async_copy(src_hbm.at[sid, pl.ds(i * CH, CH)], src_cv.at[p], isems.at[p])
        pltpu.async_copy(dst_hbm.at[sid, pl.ds(i * CH, CH)], dst_cv.at[p], isems.at[p])

    def idx_wait(i, p):
        pltpu.make_async_copy(src_hbm.at[sid, pl.ds(i * CH, CH)], src_cv.at[p], isems.at[p]).wait()
        pltpu.make_async_copy(dst_hbm.at[sid, pl.ds(i * CH, CH)], dst_cv.at[p], isems.at[p]).wait()

    idx_load(0, 0)

    def gath(p, jj, b):
        return pltpu.make_async_copy(tab_sh.at[src_cv.at[p, jj]], rows_v.at[b], gsems.at[b])

    def scat(p, jj, b):
        return pltpu.make_async_copy(rows_v.at[b], acc_sh.at[dst_cv.at[p, jj]], ssems.at[b])

    NCHUNK = JG // CH

    def body(i, carry):
        p = lax.rem(i, 2)
        idx_wait(i, p)

        for b in range(NB):
            @pl.when(i > 0)
            def _():
                scat(1 - p, CH - NB + b, b).wait()

            gath(p, b, b).start()

        @pl.when(i < NCHUNK - 1)
        def _():
            idx_load(i + 1, 1 - p)

        for q in range(CH // NB):
            for b in range(NB):
                jj = NB * q + b
                gath(p, jj, b).wait()
                pltpu.async_copy(rows_v.at[b], acc_sh.at[dst_cv.at[p, jj]],
                                 ssems.at[b], add=True)
            if q < CH // NB - 1:
                for b in range(NB):
                    jj = NB * q + b
                    scat(p, jj, b).wait()
                    gath(p, jj + NB, b).start()
        return carry

    lax.fori_loop(0, NCHUNK, body, 0)
    for b in range(NB):
        scat(lax.rem(NCHUNK - 1, 2), CH - NB + b, b).wait()
    plsc.subcore_barrier()

    for g in range(5):
        sl = pl.ds(zbase + g * 128, 128)
        pltpu.async_copy(acc_sh.at[sl], rows_v.at[g], gsems.at[g])
    for g in range(5):
        sl = pl.ds(zbase + g * 128, 128)
        pltpu.make_async_copy(acc_sh.at[sl], rows_v.at[g], gsems.at[g]).wait()

        @pl.when(cid == 0)
        def _():
            pltpu.async_copy(rows_v.at[g], out0.at[sl], ssems.at[g])

        @pl.when(cid == 1)
        def _():
            pltpu.async_copy(rows_v.at[g], out1.at[sl], ssems.at[g])
    for g in range(5):
        sl = pl.ds(zbase + g * 128, 128)

        @pl.when(cid == 0)
        def _():
            pltpu.make_async_copy(rows_v.at[g], out0.at[sl], ssems.at[g]).wait()

        @pl.when(cid == 1)
        def _():
            pltpu.make_async_copy(rows_v.at[g], out1.at[sl], ssems.at[g]).wait()


@functools.cache
def _make_agg():
    return functools.partial(
        pl.kernel,
        out_type=(jax.ShapeDtypeStruct((NPAD, DH), jnp.bfloat16),
                  jax.ShapeDtypeStruct((NPAD, DH), jnp.bfloat16)),
        mesh=plsc.VectorSubcoreMesh(core_axis_name="c", subcore_axis_name="s",
                                    num_cores=NC, num_subcores=NS),
        compiler_params=pltpu.CompilerParams(use_tc_tiling_on_sc=False),
        scratch_types=[
            pltpu.VMEM((2, CH, 128), jnp.int32),
            pltpu.VMEM((2, CH, 128), jnp.int32),
            pltpu.VMEM((NB, 128, DH), jnp.bfloat16),
            pltpu.VMEM_SHARED((NPAD, DH), jnp.bfloat16),
            pltpu.VMEM_SHARED((NPAD, DH), jnp.bfloat16),
            pltpu.SemaphoreType.DMA((NB,)),
            pltpu.SemaphoreType.DMA((NB,)),
            pltpu.SemaphoreType.DMA((2,)),
        ],
    )(_agg_body)


def _agg(*args):
    return _make_agg()(*args)


R = 2000
NBLK = N // R


def _mlp_mid_body(a0_ref, a1_ref, w1_ref, b1_ref, w2_ref, b2_ref, hp_ref):
    z = jnp.concatenate([a0_ref[...], a1_ref[...]], axis=1).astype(jnp.float32)
    t = jnp.dot(z, w1_ref[...], preferred_element_type=jnp.float32) + b1_ref[...]
    t = jnp.maximum(t, 0.0)
    h = jnp.dot(t, w2_ref[...], preferred_element_type=jnp.float32) + b2_ref[...]
    hb = jnp.maximum(h, 0.0).astype(jnp.bfloat16)
    hp_ref[...] = jnp.stack([hb[:, :DH], hb[:, DH:]])


_W_SPECS = [pl.BlockSpec((D, D), lambda i: (0, 0)),
            pl.BlockSpec((1, D), lambda i: (0, 0)),
            pl.BlockSpec((D, D), lambda i: (0, 0)),
            pl.BlockSpec((1, D), lambda i: (0, 0))]


def _mlp_mid(a0, a1, W1, b1, W2, b2, *, interpret=False):
    return pl.pallas_call(
        _mlp_mid_body,
        grid=(NBLK,),
        in_specs=[pl.BlockSpec((R, DH), lambda i: (i, 0)),
                  pl.BlockSpec((R, DH), lambda i: (i, 0))] + _W_SPECS,
        out_specs=pl.BlockSpec((NC, R, DH), lambda i: (0, i, 0)),
        out_shape=jax.ShapeDtypeStruct((NC, NPAD, DH), jnp.bfloat16),
        interpret=interpret,
    )(a0, a1, W1, b1.reshape(1, D), W2, b2.reshape(1, D))


def _mlp_pool_body(a0_ref, a1_ref, w1_ref, b1_ref, w2_ref, b2_ref,
                   batch_ref, wl_ref, bl_ref, o_ref, sums_ref, cnt_ref):
    i = pl.program_id(0)

    @pl.when(i == 0)
    def _():
        sums_ref[...] = jnp.zeros_like(sums_ref)
        cnt_ref[...] = jnp.zeros_like(cnt_ref)

    z = jnp.concatenate([a0_ref[...], a1_ref[...]], axis=1).astype(jnp.float32)
    t = jnp.dot(z, w1_ref[...], preferred_element_type=jnp.float32) + b1_ref[...]
    t = jnp.maximum(t, 0.0)
    h = jnp.dot(t, w2_ref[...], preferred_element_type=jnp.float32) + b2_ref[...]
    h = jnp.maximum(h, 0.0)
    gids = lax.broadcasted_iota(jnp.int32, (G, R), 0)
    mask = (gids == batch_ref[0]).astype(jnp.float32)
    sums_ref[...] += jnp.dot(mask, h, preferred_element_type=jnp.float32)
    cnt_ref[...] += jnp.sum(mask, axis=1, keepdims=True)

    @pl.when(i == NBLK - 1)
    def _():
        mean = sums_ref[...] / jnp.maximum(cnt_ref[...], 1.0)
        p = jnp.dot(mean, wl_ref[...], preferred_element_type=jnp.float32) + bl_ref[...]
        m = jnp.max(p, axis=1, keepdims=True)
        lse = m + jnp.log(jnp.sum(jnp.exp(p - m), axis=1, keepdims=True))
        o_ref[...] = p - lse


def _mlp_pool(a0, a1, W1, b1, W2, b2, batch3d, Wl, bl, *, interpret=False):
    return pl.pallas_call(
        _mlp_pool_body,
        grid=(NBLK,),
        in_specs=[pl.BlockSpec((R, DH), lambda i: (i, 0)),
                  pl.BlockSpec((R, DH), lambda i: (i, 0))] + _W_SPECS + [
                  pl.BlockSpec((1, 1, R), lambda i: (i, 0, 0)),
                  pl.BlockSpec((D, C), lambda i: (0, 0)),
                  pl.BlockSpec((1, C), lambda i: (0, 0))],
        out_specs=pl.BlockSpec((G, C), lambda i: (0, 0)),
        out_shape=jax.ShapeDtypeStruct((G, C), jnp.float32),
        scratch_shapes=[pltpu.VMEM((G, D), jnp.float32),
                        pltpu.VMEM((G, 1), jnp.float32)],
        interpret=interpret,
    )(a0, a1, W1, b1.reshape(1, D), W2, b2.reshape(1, D),
      batch3d, Wl, bl.reshape(1, C))


def kernel(x, edge_index, batch, W1a, b1a, W2a, b2a, W1b, b1b, W2b, b2b, Wl, bl):
    src = edge_index[0]
    dst = edge_index[1]
    pad = E_PAD - E
    srcp = jnp.concatenate([src, jnp.zeros((pad,), jnp.int32)]).reshape(NS, JG, 128)
    dstp = jnp.concatenate([dst, jnp.full((pad,), N, jnp.int32)]).reshape(NS, JG, 128)

    xb = x.astype(jnp.bfloat16)
    xp = jnp.zeros((NC, NPAD, DH), jnp.bfloat16)
    xp = xp.at[0, :N].set(xb[:, :DH]).at[1, :N].set(xb[:, DH:])

    a0, a1 = _agg(xp, srcp, dstp)
    hp1 = _mlp_mid(a0, a1, W1a, b1a, W2a, b2a)
    a0, a1 = _agg(hp1, srcp, dstp)
    return _mlp_pool(a0, a1, W1b, b1b, W2b, b2b, batch.reshape(NBLK, 1, R), Wl, bl)
